# baseline (speedup 1.0000x reference)
import functools

import jax
import jax.numpy as jnp
from jax import lax
from jax.experimental import pallas as pl
from jax.experimental.pallas import tpu as pltpu
from jax.experimental.pallas import tpu_sc as plsc

N = 10000
E = 320000
IN_DIM = 128
HID = 256
OUT = 128

NC = 2
NS = 16
C = 128
CPT = 156
EPT = C * CPT
HW = IN_DIM // 2
NP = 10240
RPT = NP // NS

BR = 1280
GRID = NP // BR

_MESH = dict(core_axis_name="c", subcore_axis_name="s", num_cores=NC,
             num_subcores=NS)


def _deg_body(g_hbm, out_hbm, ring_v, ones_v, zer_v, deg_sh, isem0, isem1,
              asem):
    c = lax.axis_index("c")
    s = lax.axis_index("s")
    base = s * EPT
    npairs = jnp.where(s == NS - 1, 80, 78)
    nch = 2 * npairs

    def fill_ones(i, _):
        ones_v[pl.ds(i * 16, 16)] = jnp.full((16,), 1.0, jnp.float32)
        return 0
    lax.fori_loop(0, C // 16, fill_ones, 0)

    def fill_zero(i, _):
        zer_v[pl.ds(i * 16, 16)] = jnp.zeros((16,), jnp.float32)
        return 0
    lax.fori_loop(0, RPT // 16, fill_zero, 0)

    pltpu.sync_copy(zer_v, deg_sh.at[pl.ds(s * RPT, RPT)])
    plsc.subcore_barrier()

    pltpu.async_copy(g_hbm.at[c, pl.ds(base, C)], ring_v.at[0], isem0)

    def pair(k, _):
        j0 = 2 * k
        pltpu.async_copy(g_hbm.at[c, pl.ds(base + (j0 + 1) * C, C)],
                         ring_v.at[1], isem1)
        pltpu.make_async_copy(g_hbm.at[c, pl.ds(base + j0 * C, C)],
                              ring_v.at[0], isem0).wait()
        pltpu.sync_copy(ones_v, deg_sh.at[ring_v.at[0]], add=True)

        @pl.when(j0 + 2 < nch)
        def _():
            pltpu.async_copy(g_hbm.at[c, pl.ds(base + (j0 + 2) * C, C)],
                             ring_v.at[0], isem0)

        pltpu.make_async_copy(g_hbm.at[c, pl.ds(base + (j0 + 1) * C, C)],
                              ring_v.at[1], isem1).wait()
        pltpu.sync_copy(ones_v, deg_sh.at[ring_v.at[1]], add=True)
        return 0
    lax.fori_loop(0, npairs, pair, 0)
    plsc.subcore_barrier()

    pltpu.sync_copy(deg_sh.at[pl.ds(s * RPT, RPT)],
                    out_hbm.at[c, pl.ds(s * RPT, RPT)])


@functools.cache
def _deg_call():
    return pl.kernel(
        _deg_body,
        out_type=jax.ShapeDtypeStruct((NC, NP), jnp.float32),
        mesh=plsc.VectorSubcoreMesh(**_MESH),
        scratch_types=[
            pltpu.VMEM((2, C), jnp.int32),
            pltpu.VMEM((C,), jnp.float32),
            pltpu.VMEM((RPT,), jnp.float32),
            pltpu.VMEM_SHARED((NP,), jnp.float32),
            pltpu.SemaphoreType.DMA,
            pltpu.SemaphoreType.DMA,
            pltpu.SemaphoreType.DMA,
        ],
    )


def _agg_body(tab_hbm, g_hbm, out_hbm, src_v, dring_v, buf0_v, buf1_v,
              acc_sh, sem0, sem1, isem0, isem1):
    c = lax.axis_index("c")
    s = lax.axis_index("s")
    base = s * EPT
    npairs = jnp.where(s == NS - 1, 80, 78)
    nch = 2 * npairs
    tab = tab_hbm.at[c]

    pltpu.sync_copy(g_hbm.at[0, pl.ds(base, C * 160)], src_v)

    def zrow(r, _):
        def zcol(k, _):
            buf0_v[r, pl.ds(k * 16, 16)] = jnp.zeros((16,), jnp.float32)
            return 0
        lax.fori_loop(0, HW // 16, zcol, 0)
        return 0
    lax.fori_loop(0, C, zrow, 0)

    def zcp(k, _):
        pltpu.sync_copy(buf0_v, acc_sh.at[pl.ds(s * RPT + k * C, C)])
        return 0
    lax.fori_loop(0, RPT // C, zcp, 0)
    plsc.subcore_barrier()

    pltpu.async_copy(g_hbm.at[1, pl.ds(base, C)], dring_v.at[0], isem0)
    pltpu.async_copy(tab.at[src_v.at[pl.ds(0, C)]], buf0_v, sem0)

    def pair(k, _):
        j0 = 2 * k
        pltpu.async_copy(tab.at[src_v.at[pl.ds((j0 + 1) * C, C)]], buf1_v,
                         sem1)
        pltpu.async_copy(g_hbm.at[1, pl.ds(base + (j0 + 1) * C, C)],
                         dring_v.at[1], isem1)
        pltpu.make_async_copy(g_hbm.at[1, pl.ds(base + j0 * C, C)],
                              dring_v.at[0], isem0).wait()
        pltpu.make_async_copy(tab.at[src_v.at[pl.ds(j0 * C, C)]], buf0_v,
                              sem0).wait()
        pltpu.sync_copy(buf0_v, acc_sh.at[dring_v.at[0]], add=True)

        @pl.when(j0 + 2 < nch)
        def _():
            pltpu.async_copy(tab.at[src_v.at[pl.ds((j0 + 2) * C, C)]],
                             buf0_v, sem0)
            pltpu.async_copy(g_hbm.at[1, pl.ds(base + (j0 + 2) * C, C)],
                             dring_v.at[0], isem0)

        pltpu.make_async_copy(g_hbm.at[1, pl.ds(base + (j0 + 1) * C, C)],
                              dring_v.at[1], isem1).wait()
        pltpu.make_async_copy(tab.at[src_v.at[pl.ds((j0 + 1) * C, C)]],
                              buf1_v, sem1).wait()
        pltpu.sync_copy(buf1_v, acc_sh.at[dring_v.at[1]], add=True)
        return 0
    lax.fori_loop(0, npairs, pair, 0)
    plsc.subcore_barrier()

    def cout(k, _):
        pltpu.sync_copy(acc_sh.at[pl.ds(s * RPT + k * C, C)],
                        out_hbm.at[c, pl.ds(s * RPT + k * C, C)])
        return 0
    lax.fori_loop(0, RPT // C, cout, 0)


@functools.cache
def _agg_call():
    return pl.kernel(
        _agg_body,
        out_type=jax.ShapeDtypeStruct((NC, NP, HW), jnp.float32),
        mesh=plsc.VectorSubcoreMesh(**_MESH),
        scratch_types=[
            pltpu.VMEM((C * 160,), jnp.int32),
            pltpu.VMEM((2, C), jnp.int32),
            pltpu.VMEM((C, HW), jnp.float32),
            pltpu.VMEM((C, HW), jnp.float32),
            pltpu.VMEM_SHARED((NP, HW), jnp.float32),
            pltpu.SemaphoreType.DMA,
            pltpu.SemaphoreType.DMA,
            pltpu.SemaphoreType.DMA,
            pltpu.SemaphoreType.DMA,
        ],
        compiler_params=pltpu.CompilerParams(use_tc_tiling_on_sc=False),
    )


def _norm_m1_body(deg_ref, feat_ref, m1_ref, ns_ref, nd_ref):
    d = deg_ref[...]
    deg_o = d[0]
    deg_i = d[1]
    ns = jnp.where(deg_o > 0, lax.rsqrt(jnp.maximum(deg_o, 1.0)), 0.0)
    nd = jnp.where(deg_i > 0, lax.rsqrt(jnp.maximum(deg_i, 1.0)), 0.0)
    ns_ref[...] = ns[:, None]
    nd_ref[...] = nd[:, None]
    f = feat_ref[...] * ns[:, None]
    m1_ref[0] = f[:, :HW]
    m1_ref[1] = f[:, HW:]


_norm_m1_call = pl.pallas_call(
    _norm_m1_body,
    grid=(GRID,),
    in_specs=[
        pl.BlockSpec((2, BR), lambda i: (0, i)),
        pl.BlockSpec((BR, IN_DIM), lambda i: (i, 0)),
    ],
    out_specs=[
        pl.BlockSpec((2, BR, HW), lambda i: (0, i, 0)),
        pl.BlockSpec((BR, 1), lambda i: (i, 0)),
        pl.BlockSpec((BR, 1), lambda i: (i, 0)),
    ],
    out_shape=[
        jax.ShapeDtypeStruct((2, NP, HW), jnp.float32),
        jax.ShapeDtypeStruct((NP, 1), jnp.float32),
        jax.ShapeDtypeStruct((NP, 1), jnp.float32),
    ],
)


def _mid_body(q_ref, ns_ref, nd_ref, w1_ref, b1_ref, w2_ref, h1_ref, m2_ref):
    a = jnp.concatenate([q_ref[0], q_ref[1]], axis=1) * nd_ref[...]
    h1 = jnp.dot(a, w1_ref[...], preferred_element_type=jnp.float32)
    h1 = jnp.maximum(h1 + b1_ref[...], 0.0)
    h1_ref[...] = h1
    p = jnp.dot(h1, w2_ref[...], preferred_element_type=jnp.float32)
    m2 = p * ns_ref[...]
    m2_ref[0] = m2[:, :HW]
    m2_ref[1] = m2[:, HW:]


_mid_call = pl.pallas_call(
    _mid_body,
    grid=(GRID,),
    in_specs=[
        pl.BlockSpec((2, BR, HW), lambda i: (0, i, 0)),
        pl.BlockSpec((BR, 1), lambda i: (i, 0)),
        pl.BlockSpec((BR, 1), lambda i: (i, 0)),
        pl.BlockSpec((IN_DIM, HID), lambda i: (0, 0)),
        pl.BlockSpec((1, HID), lambda i: (0, 0)),
        pl.BlockSpec((HID, OUT), lambda i: (0, 0)),
    ],
    out_specs=[
        pl.BlockSpec((BR, HID), lambda i: (i, 0)),
        pl.BlockSpec((2, BR, HW), lambda i: (0, i, 0)),
    ],
    out_shape=[
        jax.ShapeDtypeStruct((N, HID), jnp.float32),
        jax.ShapeDtypeStruct((2, NP, HW), jnp.float32),
    ],
)


def _fin_body(q_ref, nd_ref, b2_ref, cls_ref, ws_ref, h2_ref, sig_ref):
    a = jnp.concatenate([q_ref[0], q_ref[1]], axis=1) * nd_ref[...]
    h2_ref[...] = a + b2_ref[...]

    @pl.when(pl.program_id(0) == 0)
    def _():
        z = jnp.dot(cls_ref[...], ws_ref[...],
                    preferred_element_type=jnp.float32)
        sig_ref[...] = jnp.maximum(z, 0.0) + jnp.log(1.0 + jnp.exp(-jnp.abs(z)))


_fin_call = pl.pallas_call(
    _fin_body,
    grid=(GRID,),
    in_specs=[
        pl.BlockSpec((2, BR, HW), lambda i: (0, i, 0)),
        pl.BlockSpec((BR, 1), lambda i: (i, 0)),
        pl.BlockSpec((1, OUT), lambda i: (0, 0)),
        pl.BlockSpec((10, IN_DIM), lambda i: (0, 0)),
        pl.BlockSpec((IN_DIM, 1), lambda i: (0, 0)),
    ],
    out_specs=[
        pl.BlockSpec((BR, OUT), lambda i: (i, 0)),
        pl.BlockSpec((10, 1), lambda i: (0, 0)),
    ],
    out_shape=[
        jax.ShapeDtypeStruct((N, OUT), jnp.float32),
        jax.ShapeDtypeStruct((10, 1), jnp.float32),
    ],
)


def kernel(g, feat, cls_spec_avg_feats, W1, b1, W2, b2, W_sigma):
    deg = _deg_call()(g)
    m1, ns, nd = _norm_m1_call(deg, feat)

    q1 = _agg_call()(m1, g)
    h1o, m2 = _mid_call(q1, ns, nd, W1, b1.reshape(1, HID), W2)

    q2 = _agg_call()(m2, g)
    h2o, sig2 = _fin_call(q2, nd, b2.reshape(1, OUT), cls_spec_avg_feats,
                          W_sigma)
    return (h2o, h1o, h2o, sig2[:, 0])


# speedup vs baseline: 1.2308x; 1.2308x over previous
import functools

import jax
import jax.numpy as jnp
from jax import lax
from jax.experimental import pallas as pl
from jax.experimental.pallas import tpu as pltpu
from jax.experimental.pallas import tpu_sc as plsc

N = 10000
E = 320000
IN_DIM = 128
HID = 256
OUT = 128

NC = 2
NS = 16
NW = NC * NS
C = 128
NP = 10240
RPT = NP // NS

A_CPT = 78
A_EPT = C * A_CPT
A_LAST = 82

D_CPT = 156
D_EPT = C * D_CPT
D_LAST = 160

BR = 1280
GRID = NP // BR

_MESH = dict(core_axis_name="c", subcore_axis_name="s", num_cores=NC,
             num_subcores=NS)


def _deg_body(g_hbm, out_hbm, ring_v, ones_v, zer_v, deg_sh, isem0, isem1,
              asem):
    c = lax.axis_index("c")
    s = lax.axis_index("s")
    base = s * D_EPT
    npairs = jnp.where(s == NS - 1, D_LAST // 2, D_CPT // 2)
    nch = 2 * npairs

    def fill_ones(i, _):
        ones_v[pl.ds(i * 16, 16)] = jnp.full((16,), 1.0, jnp.float32)
        return 0
    lax.fori_loop(0, C // 16, fill_ones, 0)

    def fill_zero(i, _):
        zer_v[pl.ds(i * 16, 16)] = jnp.zeros((16,), jnp.float32)
        return 0
    lax.fori_loop(0, RPT // 16, fill_zero, 0)

    pltpu.sync_copy(zer_v, deg_sh.at[pl.ds(s * RPT, RPT)])
    plsc.subcore_barrier()

    pltpu.async_copy(g_hbm.at[c, pl.ds(base, C)], ring_v.at[0], isem0)

    def pair(k, _):
        j0 = 2 * k
        pltpu.async_copy(g_hbm.at[c, pl.ds(base + (j0 + 1) * C, C)],
                         ring_v.at[1], isem1)
        pltpu.make_async_copy(g_hbm.at[c, pl.ds(base + j0 * C, C)],
                              ring_v.at[0], isem0).wait()
        pltpu.sync_copy(ones_v, deg_sh.at[ring_v.at[0]], add=True)

        @pl.when(j0 + 2 < nch)
        def _():
            pltpu.async_copy(g_hbm.at[c, pl.ds(base + (j0 + 2) * C, C)],
                             ring_v.at[0], isem0)

        pltpu.make_async_copy(g_hbm.at[c, pl.ds(base + (j0 + 1) * C, C)],
                              ring_v.at[1], isem1).wait()
        pltpu.sync_copy(ones_v, deg_sh.at[ring_v.at[1]], add=True)
        return 0
    lax.fori_loop(0, npairs, pair, 0)
    plsc.subcore_barrier()

    pltpu.sync_copy(deg_sh.at[pl.ds(s * RPT, RPT)],
                    out_hbm.at[c, pl.ds(s * RPT, RPT)])


@functools.cache
def _deg_call():
    return pl.kernel(
        _deg_body,
        out_type=jax.ShapeDtypeStruct((NC, NP), jnp.float32),
        mesh=plsc.VectorSubcoreMesh(**_MESH),
        scratch_types=[
            pltpu.VMEM((2, C), jnp.int32),
            pltpu.VMEM((C,), jnp.float32),
            pltpu.VMEM((RPT,), jnp.float32),
            pltpu.VMEM_SHARED((NP,), jnp.float32),
            pltpu.SemaphoreType.DMA,
            pltpu.SemaphoreType.DMA,
            pltpu.SemaphoreType.DMA,
        ],
    )


def _agg_body(tab_hbm, g_hbm, out_hbm, src_v, dring_v, buf0_v, buf1_v,
              acc_sh, sem0, sem1, isem0, isem1):
    c = lax.axis_index("c")
    s = lax.axis_index("s")
    wid = c * NS + s
    base = wid * A_EPT
    npairs = jnp.where(wid == NW - 1, A_LAST // 2, A_CPT // 2)
    nch = 2 * npairs

    pltpu.sync_copy(g_hbm.at[0, pl.ds(base, C * A_LAST)], src_v)

    def zrow(r, _):
        def zcol(k, _):
            buf0_v[r, pl.ds(k * 16, 16)] = jnp.zeros((16,), jnp.float32)
            return 0
        lax.fori_loop(0, IN_DIM // 16, zcol, 0)
        return 0
    lax.fori_loop(0, C, zrow, 0)

    def zcp(k, _):
        pltpu.sync_copy(buf0_v, acc_sh.at[pl.ds(s * RPT + k * C, C)])
        return 0
    lax.fori_loop(0, RPT // C, zcp, 0)
    plsc.subcore_barrier()

    pltpu.async_copy(g_hbm.at[1, pl.ds(base, C)], dring_v.at[0], isem0)
    pltpu.async_copy(tab_hbm.at[src_v.at[pl.ds(0, C)]], buf0_v, sem0)

    def pair(k, _):
        j0 = 2 * k
        pltpu.async_copy(tab_hbm.at[src_v.at[pl.ds((j0 + 1) * C, C)]],
                         buf1_v, sem1)
        pltpu.async_copy(g_hbm.at[1, pl.ds(base + (j0 + 1) * C, C)],
                         dring_v.at[1], isem1)
        pltpu.make_async_copy(g_hbm.at[1, pl.ds(base + j0 * C, C)],
                              dring_v.at[0], isem0).wait()
        pltpu.make_async_copy(tab_hbm.at[src_v.at[pl.ds(j0 * C, C)]],
                              buf0_v, sem0).wait()
        pltpu.sync_copy(buf0_v, acc_sh.at[dring_v.at[0]], add=True)

        @pl.when(j0 + 2 < nch)
        def _():
            pltpu.async_copy(tab_hbm.at[src_v.at[pl.ds((j0 + 2) * C, C)]],
                             buf0_v, sem0)
            pltpu.async_copy(g_hbm.at[1, pl.ds(base + (j0 + 2) * C, C)],
                             dring_v.at[0], isem0)

        pltpu.make_async_copy(g_hbm.at[1, pl.ds(base + (j0 + 1) * C, C)],
                              dring_v.at[1], isem1).wait()
        pltpu.make_async_copy(tab_hbm.at[src_v.at[pl.ds((j0 + 1) * C, C)]],
                              buf1_v, sem1).wait()
        pltpu.sync_copy(buf1_v, acc_sh.at[dring_v.at[1]], add=True)
        return 0
    lax.fori_loop(0, npairs, pair, 0)
    plsc.subcore_barrier()

    def cout(k, _):
        pltpu.sync_copy(acc_sh.at[pl.ds(s * RPT + k * C, C)],
                        out_hbm.at[c, pl.ds(s * RPT + k * C, C)])
        return 0
    lax.fori_loop(0, RPT // C, cout, 0)


@functools.cache
def _agg_call():
    return pl.kernel(
        _agg_body,
        out_type=jax.ShapeDtypeStruct((NC, NP, IN_DIM), jnp.float32),
        mesh=plsc.VectorSubcoreMesh(**_MESH),
        scratch_types=[
            pltpu.VMEM((C * A_LAST,), jnp.int32),
            pltpu.VMEM((2, C), jnp.int32),
            pltpu.VMEM((C, IN_DIM), jnp.float32),
            pltpu.VMEM((C, IN_DIM), jnp.float32),
            pltpu.VMEM_SHARED((NP, IN_DIM), jnp.float32),
            pltpu.SemaphoreType.DMA,
            pltpu.SemaphoreType.DMA,
            pltpu.SemaphoreType.DMA,
            pltpu.SemaphoreType.DMA,
        ],
    )


def _norm_m1_body(deg_ref, feat_ref, m1_ref, ns_ref, nd_ref):
    d = deg_ref[...]
    deg_o = d[0]
    deg_i = d[1]
    ns = jnp.where(deg_o > 0, lax.rsqrt(jnp.maximum(deg_o, 1.0)), 0.0)
    nd = jnp.where(deg_i > 0, lax.rsqrt(jnp.maximum(deg_i, 1.0)), 0.0)
    ns_ref[...] = ns[:, None]
    nd_ref[...] = nd[:, None]
    m1_ref[...] = feat_ref[...] * ns[:, None]


_norm_m1_call = pl.pallas_call(
    _norm_m1_body,
    grid=(GRID,),
    in_specs=[
        pl.BlockSpec((2, BR), lambda i: (0, i)),
        pl.BlockSpec((BR, IN_DIM), lambda i: (i, 0)),
    ],
    out_specs=[
        pl.BlockSpec((BR, IN_DIM), lambda i: (i, 0)),
        pl.BlockSpec((BR, 1), lambda i: (i, 0)),
        pl.BlockSpec((BR, 1), lambda i: (i, 0)),
    ],
    out_shape=[
        jax.ShapeDtypeStruct((NP, IN_DIM), jnp.float32),
        jax.ShapeDtypeStruct((NP, 1), jnp.float32),
        jax.ShapeDtypeStruct((NP, 1), jnp.float32),
    ],
)


def _mid_body(q_ref, ns_ref, nd_ref, w1_ref, b1_ref, w2_ref, h1_ref, m2_ref):
    a = (q_ref[0] + q_ref[1]) * nd_ref[...]
    h1 = jnp.dot(a, w1_ref[...], preferred_element_type=jnp.float32)
    h1 = jnp.maximum(h1 + b1_ref[...], 0.0)
    h1_ref[...] = h1
    p = jnp.dot(h1, w2_ref[...], preferred_element_type=jnp.float32)
    m2_ref[...] = p * ns_ref[...]


_mid_call = pl.pallas_call(
    _mid_body,
    grid=(GRID,),
    in_specs=[
        pl.BlockSpec((2, BR, IN_DIM), lambda i: (0, i, 0)),
        pl.BlockSpec((BR, 1), lambda i: (i, 0)),
        pl.BlockSpec((BR, 1), lambda i: (i, 0)),
        pl.BlockSpec((IN_DIM, HID), lambda i: (0, 0)),
        pl.BlockSpec((1, HID), lambda i: (0, 0)),
        pl.BlockSpec((HID, OUT), lambda i: (0, 0)),
    ],
    out_specs=[
        pl.BlockSpec((BR, HID), lambda i: (i, 0)),
        pl.BlockSpec((BR, OUT), lambda i: (i, 0)),
    ],
    out_shape=[
        jax.ShapeDtypeStruct((N, HID), jnp.float32),
        jax.ShapeDtypeStruct((NP, OUT), jnp.float32),
    ],
)


def _fin_body(q_ref, nd_ref, b2_ref, cls_ref, ws_ref, h2_ref, sig_ref):
    h2_ref[...] = (q_ref[0] + q_ref[1]) * nd_ref[...] + b2_ref[...]

    @pl.when(pl.program_id(0) == 0)
    def _():
        z = jnp.dot(cls_ref[...], ws_ref[...],
                    preferred_element_type=jnp.float32)
        sig_ref[...] = jnp.maximum(z, 0.0) + jnp.log(1.0 + jnp.exp(-jnp.abs(z)))


_fin_call = pl.pallas_call(
    _fin_body,
    grid=(GRID,),
    in_specs=[
        pl.BlockSpec((2, BR, OUT), lambda i: (0, i, 0)),
        pl.BlockSpec((BR, 1), lambda i: (i, 0)),
        pl.BlockSpec((1, OUT), lambda i: (0, 0)),
        pl.BlockSpec((10, IN_DIM), lambda i: (0, 0)),
        pl.BlockSpec((IN_DIM, 1), lambda i: (0, 0)),
    ],
    out_specs=[
        pl.BlockSpec((BR, OUT), lambda i: (i, 0)),
        pl.BlockSpec((10, 1), lambda i: (0, 0)),
    ],
    out_shape=[
        jax.ShapeDtypeStruct((N, OUT), jnp.float32),
        jax.ShapeDtypeStruct((10, 1), jnp.float32),
    ],
)


def kernel(g, feat, cls_spec_avg_feats, W1, b1, W2, b2, W_sigma):
    deg = _deg_call()(g)
    m1, ns, nd = _norm_m1_call(deg, feat)

    q1 = _agg_call()(m1, g)
    h1o, m2 = _mid_call(q1, ns, nd, W1, b1.reshape(1, HID), W2)

    q2 = _agg_call()(m2, g)
    h2o, sig2 = _fin_call(q2, nd, b2.reshape(1, OUT), cls_spec_avg_feats,
                          W_sigma)
    return (h2o, h1o, h2o, sig2[:, 0])


# speedup vs baseline: 1.3400x; 1.0887x over previous
import functools

import jax
import jax.numpy as jnp
from jax import lax
from jax.experimental import pallas as pl
from jax.experimental.pallas import tpu as pltpu
from jax.experimental.pallas import tpu_sc as plsc

N = 10000
E = 320000
IN_DIM = 128
HID = 256
OUT = 128

NC = 2
NS = 16
NW = NC * NS
C = 128
NP = 10240
RPT = NP // NS

A_CPT = 78
A_EPT = C * A_CPT
A_LAST = 82

D_CP2 = 160
EP = NS * D_CP2 * C

BR = 1280
GRID = NP // BR

_MESH = dict(core_axis_name="c", subcore_axis_name="s", num_cores=NC,
             num_subcores=NS)


def _deg_body(gp_hbm, out_hbm, idx_v, ones_v, zer_v, deg_sh, asem0, asem1):
    c = lax.axis_index("c")
    s = lax.axis_index("s")
    pltpu.sync_copy(gp_hbm.at[c, s], idx_v)

    def fill_ones(i, _):
        ones_v[pl.ds(i * 16, 16)] = jnp.full((16,), 1.0, jnp.float32)
        return 0
    lax.fori_loop(0, C // 16, fill_ones, 0)

    def fill_zero(i, _):
        zer_v[pl.ds(i * 16, 16)] = jnp.zeros((16,), jnp.float32)
        return 0
    lax.fori_loop(0, RPT // 16, fill_zero, 0)

    pltpu.sync_copy(zer_v, deg_sh.at[pl.ds(s * RPT, RPT)])
    plsc.subcore_barrier()

    def pair(k, _):
        d0 = pltpu.async_copy(ones_v, deg_sh.at[idx_v.at[2 * k]], asem0,
                              add=True)
        d1 = pltpu.async_copy(ones_v, deg_sh.at[idx_v.at[2 * k + 1]], asem1,
                              add=True)
        d0.wait()
        d1.wait()
        return 0
    lax.fori_loop(0, D_CP2 // 2, pair, 0)
    plsc.subcore_barrier()

    pltpu.sync_copy(deg_sh.at[pl.ds(s * RPT, RPT)],
                    out_hbm.at[c, pl.ds(s * RPT, RPT)])


@functools.cache
def _deg_call():
    return pl.kernel(
        _deg_body,
        out_type=jax.ShapeDtypeStruct((NC, NP), jnp.float32),
        mesh=plsc.VectorSubcoreMesh(**_MESH),
        scratch_types=[
            pltpu.VMEM((D_CP2, C), jnp.int32),
            pltpu.VMEM((C,), jnp.float32),
            pltpu.VMEM((RPT,), jnp.float32),
            pltpu.VMEM_SHARED((NP,), jnp.float32),
            pltpu.SemaphoreType.DMA,
            pltpu.SemaphoreType.DMA,
        ],
    )


def _agg_body(tab_hbm, g_hbm, out_hbm, src_v, dring_v, buf0_v, buf1_v,
              acc_sh, sem0, sem1, isem0, isem1):
    c = lax.axis_index("c")
    s = lax.axis_index("s")
    wid = c * NS + s
    base = wid * A_EPT
    npairs = jnp.where(wid == NW - 1, A_LAST // 2, A_CPT // 2)
    nch = 2 * npairs

    pltpu.sync_copy(g_hbm.at[0, pl.ds(base, C * A_LAST)], src_v)

    def zrow(r, _):
        def zcol(k, _):
            buf0_v[r, pl.ds(k * 16, 16)] = jnp.zeros((16,), jnp.float32)
            return 0
        lax.fori_loop(0, IN_DIM // 16, zcol, 0)
        return 0
    lax.fori_loop(0, C, zrow, 0)

    def zcp(k, _):
        pltpu.sync_copy(buf0_v, acc_sh.at[pl.ds(s * RPT + k * C, C)])
        return 0
    lax.fori_loop(0, RPT // C, zcp, 0)
    plsc.subcore_barrier()

    pltpu.async_copy(g_hbm.at[1, pl.ds(base, C)], dring_v.at[0], isem0)
    pltpu.async_copy(tab_hbm.at[src_v.at[pl.ds(0, C)]], buf0_v, sem0)

    def pair(k, _):
        j0 = 2 * k
        pltpu.async_copy(tab_hbm.at[src_v.at[pl.ds((j0 + 1) * C, C)]],
                         buf1_v, sem1)
        pltpu.async_copy(g_hbm.at[1, pl.ds(base + (j0 + 1) * C, C)],
                         dring_v.at[1], isem1)
        pltpu.make_async_copy(g_hbm.at[1, pl.ds(base + j0 * C, C)],
                              dring_v.at[0], isem0).wait()
        pltpu.make_async_copy(tab_hbm.at[src_v.at[pl.ds(j0 * C, C)]],
                              buf0_v, sem0).wait()
        pltpu.sync_copy(buf0_v, acc_sh.at[dring_v.at[0]], add=True)

        @pl.when(j0 + 2 < nch)
        def _():
            pltpu.async_copy(tab_hbm.at[src_v.at[pl.ds((j0 + 2) * C, C)]],
                             buf0_v, sem0)
            pltpu.async_copy(g_hbm.at[1, pl.ds(base + (j0 + 2) * C, C)],
                             dring_v.at[0], isem0)

        pltpu.make_async_copy(g_hbm.at[1, pl.ds(base + (j0 + 1) * C, C)],
                              dring_v.at[1], isem1).wait()
        pltpu.make_async_copy(tab_hbm.at[src_v.at[pl.ds((j0 + 1) * C, C)]],
                              buf1_v, sem1).wait()
        pltpu.sync_copy(buf1_v, acc_sh.at[dring_v.at[1]], add=True)
        return 0
    lax.fori_loop(0, npairs, pair, 0)
    plsc.subcore_barrier()

    def cout(k, _):
        pltpu.sync_copy(acc_sh.at[pl.ds(s * RPT + k * C, C)],
                        out_hbm.at[c, pl.ds(s * RPT + k * C, C)])
        return 0
    lax.fori_loop(0, RPT // C, cout, 0)


@functools.cache
def _agg_call():
    return pl.kernel(
        _agg_body,
        out_type=jax.ShapeDtypeStruct((NC, NP, IN_DIM), jnp.float32),
        mesh=plsc.VectorSubcoreMesh(**_MESH),
        scratch_types=[
            pltpu.VMEM((C * A_LAST,), jnp.int32),
            pltpu.VMEM((2, C), jnp.int32),
            pltpu.VMEM((C, IN_DIM), jnp.float32),
            pltpu.VMEM((C, IN_DIM), jnp.float32),
            pltpu.VMEM_SHARED((NP, IN_DIM), jnp.float32),
            pltpu.SemaphoreType.DMA,
            pltpu.SemaphoreType.DMA,
            pltpu.SemaphoreType.DMA,
            pltpu.SemaphoreType.DMA,
        ],
    )


def _norm_m1_body(deg_ref, feat_ref, m1_ref, ns_ref, nd_ref):
    d = deg_ref[...]
    deg_o = d[0]
    deg_i = d[1]
    ns = jnp.where(deg_o > 0, lax.rsqrt(jnp.maximum(deg_o, 1.0)), 0.0)
    nd = jnp.where(deg_i > 0, lax.rsqrt(jnp.maximum(deg_i, 1.0)), 0.0)
    ns_ref[...] = ns[:, None]
    nd_ref[...] = nd[:, None]
    m1_ref[...] = feat_ref[...] * ns[:, None]


_norm_m1_call = pl.pallas_call(
    _norm_m1_body,
    grid=(GRID,),
    in_specs=[
        pl.BlockSpec((2, BR), lambda i: (0, i)),
        pl.BlockSpec((BR, IN_DIM), lambda i: (i, 0)),
    ],
    out_specs=[
        pl.BlockSpec((BR, IN_DIM), lambda i: (i, 0)),
        pl.BlockSpec((BR, 1), lambda i: (i, 0)),
        pl.BlockSpec((BR, 1), lambda i: (i, 0)),
    ],
    out_shape=[
        jax.ShapeDtypeStruct((NP, IN_DIM), jnp.float32),
        jax.ShapeDtypeStruct((NP, 1), jnp.float32),
        jax.ShapeDtypeStruct((NP, 1), jnp.float32),
    ],
)


def _mid_body(q_ref, ns_ref, nd_ref, w1_ref, b1_ref, w2_ref, h1_ref, m2_ref):
    a = (q_ref[0] + q_ref[1]) * nd_ref[...]
    h1 = jnp.dot(a, w1_ref[...], preferred_element_type=jnp.float32)
    h1 = jnp.maximum(h1 + b1_ref[...], 0.0)
    h1_ref[...] = h1
    p = jnp.dot(h1, w2_ref[...], preferred_element_type=jnp.float32)
    m2_ref[...] = p * ns_ref[...]


_mid_call = pl.pallas_call(
    _mid_body,
    grid=(GRID,),
    in_specs=[
        pl.BlockSpec((2, BR, IN_DIM), lambda i: (0, i, 0)),
        pl.BlockSpec((BR, 1), lambda i: (i, 0)),
        pl.BlockSpec((BR, 1), lambda i: (i, 0)),
        pl.BlockSpec((IN_DIM, HID), lambda i: (0, 0)),
        pl.BlockSpec((1, HID), lambda i: (0, 0)),
        pl.BlockSpec((HID, OUT), lambda i: (0, 0)),
    ],
    out_specs=[
        pl.BlockSpec((BR, HID), lambda i: (i, 0)),
        pl.BlockSpec((BR, OUT), lambda i: (i, 0)),
    ],
    out_shape=[
        jax.ShapeDtypeStruct((N, HID), jnp.float32),
        jax.ShapeDtypeStruct((NP, OUT), jnp.float32),
    ],
)


def _fin_body(q_ref, nd_ref, b2_ref, cls_ref, ws_ref, h2_ref, sig_ref):
    h2_ref[...] = (q_ref[0] + q_ref[1]) * nd_ref[...] + b2_ref[...]

    @pl.when(pl.program_id(0) == 0)
    def _():
        z = jnp.dot(cls_ref[...], ws_ref[...],
                    preferred_element_type=jnp.float32)
        sig_ref[...] = jnp.maximum(z, 0.0) + jnp.log(1.0 + jnp.exp(-jnp.abs(z)))


_fin_call = pl.pallas_call(
    _fin_body,
    grid=(GRID,),
    in_specs=[
        pl.BlockSpec((2, BR, OUT), lambda i: (0, i, 0)),
        pl.BlockSpec((BR, 1), lambda i: (i, 0)),
        pl.BlockSpec((1, OUT), lambda i: (0, 0)),
        pl.BlockSpec((10, IN_DIM), lambda i: (0, 0)),
        pl.BlockSpec((IN_DIM, 1), lambda i: (0, 0)),
    ],
    out_specs=[
        pl.BlockSpec((BR, OUT), lambda i: (i, 0)),
        pl.BlockSpec((10, 1), lambda i: (0, 0)),
    ],
    out_shape=[
        jax.ShapeDtypeStruct((N, OUT), jnp.float32),
        jax.ShapeDtypeStruct((10, 1), jnp.float32),
    ],
)


def kernel(g, feat, cls_spec_avg_feats, W1, b1, W2, b2, W_sigma):
    padv = N + (jnp.arange(EP - E, dtype=jnp.int32) % (NP - N))
    gp = jnp.concatenate([g, jnp.tile(padv, (2, 1))], axis=1)
    gp4 = gp.reshape(2, NS, D_CP2, C)

    deg = _deg_call()(gp4)
    m1, ns, nd = _norm_m1_call(deg, feat)

    q1 = _agg_call()(m1, g)
    h1o, m2 = _mid_call(q1, ns, nd, W1, b1.reshape(1, HID), W2)

    q2 = _agg_call()(m2, g)
    h2o, sig2 = _fin_call(q2, nd, b2.reshape(1, OUT), cls_spec_avg_feats,
                          W_sigma)
    return (h2o, h1o, h2o, sig2[:, 0])


# speedup vs baseline: 1.3522x; 1.0091x over previous
import functools

import jax
import jax.numpy as jnp
from jax import lax
from jax.experimental import pallas as pl
from jax.experimental.pallas import tpu as pltpu
from jax.experimental.pallas import tpu_sc as plsc

N = 10000
E = 320000
IN_DIM = 128
HID = 256
OUT = 128

NC = 2
NS = 16
NW = NC * NS
C = 128
NP = 10240
RPT = NP // NS

A_CPT = 78
A_EPT = C * A_CPT
A_LAST = 82

D_CP2 = 160
EP = NS * D_CP2 * C

ZR = 32

BR = 1280
GRID = NP // BR

_MESH = dict(core_axis_name="c", subcore_axis_name="s", num_cores=NC,
             num_subcores=NS)


def _deg_body(gp_hbm, out_hbm, idx_v, ones_v, zer_v, deg_sh, asem0, asem1):
    c = lax.axis_index("c")
    s = lax.axis_index("s")
    pltpu.sync_copy(gp_hbm.at[c, s], idx_v)

    def fill_ones(i, _):
        ones_v[pl.ds(i * 16, 16)] = jnp.full((16,), 1.0, jnp.float32)
        return 0
    lax.fori_loop(0, C // 16, fill_ones, 0)

    def fill_zero(i, _):
        zer_v[pl.ds(i * 16, 16)] = jnp.zeros((16,), jnp.float32)
        return 0
    lax.fori_loop(0, RPT // 16, fill_zero, 0)

    pltpu.sync_copy(zer_v, deg_sh.at[pl.ds(s * RPT, RPT)])
    plsc.subcore_barrier()

    def pair(k, _):
        d0 = pltpu.async_copy(ones_v, deg_sh.at[idx_v.at[2 * k]], asem0,
                              add=True)
        d1 = pltpu.async_copy(ones_v, deg_sh.at[idx_v.at[2 * k + 1]], asem1,
                              add=True)
        d0.wait()
        d1.wait()
        return 0
    lax.fori_loop(0, D_CP2 // 2, pair, 0)
    plsc.subcore_barrier()

    pltpu.sync_copy(deg_sh.at[pl.ds(s * RPT, RPT)],
                    out_hbm.at[c, pl.ds(s * RPT, RPT)])


@functools.cache
def _deg_call():
    return pl.kernel(
        _deg_body,
        out_type=jax.ShapeDtypeStruct((NC, NP), jnp.float32),
        mesh=plsc.VectorSubcoreMesh(**_MESH),
        scratch_types=[
            pltpu.VMEM((D_CP2, C), jnp.int32),
            pltpu.VMEM((C,), jnp.float32),
            pltpu.VMEM((RPT,), jnp.float32),
            pltpu.VMEM_SHARED((NP,), jnp.float32),
            pltpu.SemaphoreType.DMA,
            pltpu.SemaphoreType.DMA,
        ],
    )


def _agg_body(tab_hbm, g_hbm, out_hbm, src_v, dring_v, buf0_v, buf1_v,
              zbuf_v, acc_sh, sem0, sem1, isem0, isem1, psem):
    c = lax.axis_index("c")
    s = lax.axis_index("s")
    wid = c * NS + s
    base = wid * A_EPT
    npairs = jnp.where(wid == NW - 1, A_LAST // 2, A_CPT // 2)
    nch = 2 * npairs

    pltpu.async_copy(g_hbm.at[0, pl.ds(base, C * A_LAST)], src_v, psem)

    def zrow(r, _):
        def zcol(k, _):
            zbuf_v[r, pl.ds(k * 16, 16)] = jnp.zeros((16,), jnp.float32)
            return 0
        lax.fori_loop(0, IN_DIM // 16, zcol, 0)
        return 0
    lax.fori_loop(0, ZR, zrow, 0)
    pltpu.make_async_copy(g_hbm.at[0, pl.ds(base, C * A_LAST)], src_v,
                          psem).wait()

    pltpu.async_copy(g_hbm.at[1, pl.ds(base, C)], dring_v.at[0], isem0)
    pltpu.async_copy(tab_hbm.at[src_v.at[pl.ds(0, C)]], buf0_v, sem0)

    def zcp(k, _):
        pltpu.sync_copy(zbuf_v, acc_sh.at[pl.ds(s * RPT + k * ZR, ZR)])
        return 0
    lax.fori_loop(0, RPT // ZR, zcp, 0)
    plsc.subcore_barrier()

    def pair(k, _):
        j0 = 2 * k
        pltpu.async_copy(tab_hbm.at[src_v.at[pl.ds((j0 + 1) * C, C)]],
                         buf1_v, sem1)
        pltpu.async_copy(g_hbm.at[1, pl.ds(base + (j0 + 1) * C, C)],
                         dring_v.at[1], isem1)
        pltpu.make_async_copy(g_hbm.at[1, pl.ds(base + j0 * C, C)],
                              dring_v.at[0], isem0).wait()
        pltpu.make_async_copy(tab_hbm.at[src_v.at[pl.ds(j0 * C, C)]],
                              buf0_v, sem0).wait()
        pltpu.sync_copy(buf0_v, acc_sh.at[dring_v.at[0]], add=True)

        @pl.when(j0 + 2 < nch)
        def _():
            pltpu.async_copy(tab_hbm.at[src_v.at[pl.ds((j0 + 2) * C, C)]],
                             buf0_v, sem0)
            pltpu.async_copy(g_hbm.at[1, pl.ds(base + (j0 + 2) * C, C)],
                             dring_v.at[0], isem0)

        pltpu.make_async_copy(g_hbm.at[1, pl.ds(base + (j0 + 1) * C, C)],
                              dring_v.at[1], isem1).wait()
        pltpu.make_async_copy(tab_hbm.at[src_v.at[pl.ds((j0 + 1) * C, C)]],
                              buf1_v, sem1).wait()
        pltpu.sync_copy(buf1_v, acc_sh.at[dring_v.at[1]], add=True)
        return 0
    lax.fori_loop(0, npairs, pair, 0)
    plsc.subcore_barrier()

    def cout(k, _):
        pltpu.sync_copy(acc_sh.at[pl.ds(s * RPT + k * C, C)],
                        out_hbm.at[c, pl.ds(s * RPT + k * C, C)])
        return 0
    lax.fori_loop(0, RPT // C, cout, 0)


@functools.cache
def _agg_call():
    return pl.kernel(
        _agg_body,
        out_type=jax.ShapeDtypeStruct((NC, NP, IN_DIM), jnp.float32),
        mesh=plsc.VectorSubcoreMesh(**_MESH),
        scratch_types=[
            pltpu.VMEM((C * A_LAST,), jnp.int32),
            pltpu.VMEM((2, C), jnp.int32),
            pltpu.VMEM((C, IN_DIM), jnp.float32),
            pltpu.VMEM((C, IN_DIM), jnp.float32),
            pltpu.VMEM((ZR, IN_DIM), jnp.float32),
            pltpu.VMEM_SHARED((NP, IN_DIM), jnp.float32),
            pltpu.SemaphoreType.DMA,
            pltpu.SemaphoreType.DMA,
            pltpu.SemaphoreType.DMA,
            pltpu.SemaphoreType.DMA,
            pltpu.SemaphoreType.DMA,
        ],
    )


def _norm_m1_body(deg_ref, feat_ref, m1_ref, ns_ref, nd_ref):
    d = deg_ref[...]
    deg_o = d[0]
    deg_i = d[1]
    ns = jnp.where(deg_o > 0, lax.rsqrt(jnp.maximum(deg_o, 1.0)), 0.0)
    nd = jnp.where(deg_i > 0, lax.rsqrt(jnp.maximum(deg_i, 1.0)), 0.0)
    ns_ref[...] = ns[:, None]
    nd_ref[...] = nd[:, None]
    m1_ref[...] = feat_ref[...] * ns[:, None]


_norm_m1_call = pl.pallas_call(
    _norm_m1_body,
    grid=(GRID,),
    in_specs=[
        pl.BlockSpec((2, BR), lambda i: (0, i)),
        pl.BlockSpec((BR, IN_DIM), lambda i: (i, 0)),
    ],
    out_specs=[
        pl.BlockSpec((BR, IN_DIM), lambda i: (i, 0)),
        pl.BlockSpec((BR, 1), lambda i: (i, 0)),
        pl.BlockSpec((BR, 1), lambda i: (i, 0)),
    ],
    out_shape=[
        jax.ShapeDtypeStruct((NP, IN_DIM), jnp.float32),
        jax.ShapeDtypeStruct((NP, 1), jnp.float32),
        jax.ShapeDtypeStruct((NP, 1), jnp.float32),
    ],
)


def _mid_body(q_ref, ns_ref, nd_ref, w1_ref, b1_ref, w2_ref, h1_ref, m2_ref):
    a = (q_ref[0] + q_ref[1]) * nd_ref[...]
    h1 = jnp.dot(a, w1_ref[...], preferred_element_type=jnp.float32)
    h1 = jnp.maximum(h1 + b1_ref[...], 0.0)
    h1_ref[...] = h1
    p = jnp.dot(h1, w2_ref[...], preferred_element_type=jnp.float32)
    m2_ref[...] = p * ns_ref[...]


_mid_call = pl.pallas_call(
    _mid_body,
    grid=(GRID,),
    in_specs=[
        pl.BlockSpec((2, BR, IN_DIM), lambda i: (0, i, 0)),
        pl.BlockSpec((BR, 1), lambda i: (i, 0)),
        pl.BlockSpec((BR, 1), lambda i: (i, 0)),
        pl.BlockSpec((IN_DIM, HID), lambda i: (0, 0)),
        pl.BlockSpec((1, HID), lambda i: (0, 0)),
        pl.BlockSpec((HID, OUT), lambda i: (0, 0)),
    ],
    out_specs=[
        pl.BlockSpec((BR, HID), lambda i: (i, 0)),
        pl.BlockSpec((BR, OUT), lambda i: (i, 0)),
    ],
    out_shape=[
        jax.ShapeDtypeStruct((N, HID), jnp.float32),
        jax.ShapeDtypeStruct((NP, OUT), jnp.float32),
    ],
)


def _fin_body(q_ref, nd_ref, b2_ref, cls_ref, ws_ref, h2_ref, sig_ref):
    h2_ref[...] = (q_ref[0] + q_ref[1]) * nd_ref[...] + b2_ref[...]

    @pl.when(pl.program_id(0) == 0)
    def _():
        z = jnp.dot(cls_ref[...], ws_ref[...],
                    preferred_element_type=jnp.float32)
        sig_ref[...] = jnp.maximum(z, 0.0) + jnp.log(1.0 + jnp.exp(-jnp.abs(z)))


_fin_call = pl.pallas_call(
    _fin_body,
    grid=(GRID,),
    in_specs=[
        pl.BlockSpec((2, BR, OUT), lambda i: (0, i, 0)),
        pl.BlockSpec((BR, 1), lambda i: (i, 0)),
        pl.BlockSpec((1, OUT), lambda i: (0, 0)),
        pl.BlockSpec((10, IN_DIM), lambda i: (0, 0)),
        pl.BlockSpec((IN_DIM, 1), lambda i: (0, 0)),
    ],
    out_specs=[
        pl.BlockSpec((BR, OUT), lambda i: (i, 0)),
        pl.BlockSpec((10, 1), lambda i: (0, 0)),
    ],
    out_shape=[
        jax.ShapeDtypeStruct((N, OUT), jnp.float32),
        jax.ShapeDtypeStruct((10, 1), jnp.float32),
    ],
)


def kernel(g, feat, cls_spec_avg_feats, W1, b1, W2, b2, W_sigma):
    padv = N + (jnp.arange(EP - E, dtype=jnp.int32) % (NP - N))
    gp = jnp.concatenate([g, jnp.tile(padv, (2, 1))], axis=1)
    gp4 = gp.reshape(2, NS, D_CP2, C)

    deg = _deg_call()(gp4)
    m1, ns, nd = _norm_m1_call(deg, feat)

    q1 = _agg_call()(m1, g)
    h1o, m2 = _mid_call(q1, ns, nd, W1, b1.reshape(1, HID), W2)

    q2 = _agg_call()(m2, g)
    h2o, sig2 = _fin_call(q2, nd, b2.reshape(1, OUT), cls_spec_avg_feats,
                          W_sigma)
    return (h2o, h1o, h2o, sig2[:, 0])
